# baseline (speedup 1.0000x reference)
import functools

import jax
import jax.numpy as jnp
from jax import lax
from jax.experimental import pallas as pl
from jax.experimental.pallas import tpu as pltpu
from jax.experimental.pallas import tpu_sc as plsc

_NC = 2
_NS = 16
_NW = _NC * _NS
_CE = 80


def _proj_body(x_ref, ws_ref, wd_ref, a_ref, b_ref):
    x = x_ref[...]
    a_ref[...] = jnp.dot(x, ws_ref[...], preferred_element_type=jnp.float32)
    b_ref[...] = jnp.dot(x, wd_ref[...], preferred_element_type=jnp.float32)


def _proj(x, w1s, w1d):
    n, d = x.shape
    blk = 1000
    return pl.pallas_call(
        _proj_body,
        grid=(n // blk,),
        in_specs=[
            pl.BlockSpec((blk, d), lambda i: (i, 0)),
            pl.BlockSpec((d, d), lambda i: (0, 0)),
            pl.BlockSpec((d, d), lambda i: (0, 0)),
        ],
        out_specs=[
            pl.BlockSpec((blk, d), lambda i: (i, 0)),
            pl.BlockSpec((blk, d), lambda i: (i, 0)),
        ],
        out_shape=[jax.ShapeDtypeStruct((n, d), jnp.float32)] * 2,
    )(x, w1s, w1d)


_NBUF = 3


@functools.lru_cache(maxsize=None)
def _make_sc_gather(e_total, n_nodes, d):
    epw = e_total // _NW
    nchunks = epw // _CE
    nvec = d // 16
    mesh = plsc.VectorSubcoreMesh(core_axis_name="c", subcore_axis_name="s",
                                  num_cores=_NC, num_subcores=_NS)

    @functools.partial(
        pl.kernel,
        out_type=jax.ShapeDtypeStruct((e_total, d), jnp.float32),
        mesh=mesh,
        scratch_types=[
            pltpu.VMEM((epw,), jnp.int32),
            pltpu.VMEM((epw,), jnp.int32),
            pltpu.VMEM((_NBUF, _CE, d), jnp.float32),
            pltpu.VMEM((_NBUF, _CE, d), jnp.float32),
            pltpu.SemaphoreType.DMA((_NBUF,)),
            pltpu.SemaphoreType.DMA((_NBUF,)),
        ],
    )
    def sc_fn(a_hbm, b_hbm, src_hbm, dst_hbm, u_hbm,
              src_v, dst_v, wa_v, wb_v, gsem, wsem):
        wid = lax.axis_index("s") * _NC + lax.axis_index("c")
        base = wid * epw

        pltpu.sync_copy(src_hbm.at[pl.ds(base, epw)], src_v)
        pltpu.sync_copy(dst_hbm.at[pl.ds(base, epw)], dst_v)

        def gather_ops(c, s):
            io = pl.multiple_of(c * _CE, 8)
            return (
                pltpu.make_async_copy(
                    a_hbm.at[src_v.at[pl.ds(io, _CE)]], wa_v.at[s], gsem.at[s]),
                pltpu.make_async_copy(
                    b_hbm.at[dst_v.at[pl.ds(io, _CE)]], wb_v.at[s], gsem.at[s]),
            )

        def start_gather(c, s):
            for cp in gather_ops(c, s):
                cp.start()

        def wait_gather(c, s):
            for cp in gather_ops(c, s):
                cp.wait()

        def wb_op(c, s):
            off = pl.multiple_of(base + c * _CE, 8)
            return pltpu.make_async_copy(
                wa_v.at[s], u_hbm.at[pl.ds(off, _CE)], wsem.at[s])

        start_gather(0, 0)
        start_gather(1, 1)

        def chunk_body(c, carry):
            s = lax.rem(c, _NBUF)
            wait_gather(c, s)

            def row_body(r, rc):
                for v in range(nvec):
                    sl = pl.ds(v * 16, 16)
                    wa_v[s, r, sl] = wa_v[s, r, sl] + wb_v[s, r, sl]
                return rc

            lax.fori_loop(0, _CE, row_body, 0)
            wb_op(c, s).start()

            s2 = lax.rem(c + 2, _NBUF)

            @pl.when(c + 2 < nchunks)
            def _():
                @pl.when(c >= 1)
                def _():
                    wb_op(c - 1, s2).wait()
                start_gather(c + 2, s2)

            return carry

        lax.fori_loop(0, nchunks, chunk_body, 0)

        for c in (nchunks - 3, nchunks - 2, nchunks - 1):
            wb_op(c, c % _NBUF).wait()

    return sc_fn


def _final_body(u_ref, e_ref, x_ref, w1e_ref, b1_ref,
                w2x_ref, w2m_ref, b2_ref, o_ref, *, nb, k, d):
    u = (u_ref[...]
         + jnp.dot(e_ref[...], w1e_ref[...], preferred_element_type=jnp.float32)
         + b1_ref[...])
    mess = jnp.where(u > 0, u, jnp.exp(jnp.minimum(u, 0.0)) - 1.0)
    am = jnp.mean(mess.reshape(nb, k, d), axis=1)
    o_ref[...] = (jnp.dot(x_ref[...], w2x_ref[...],
                          preferred_element_type=jnp.float32)
                  + jnp.dot(am, w2m_ref[...],
                            preferred_element_type=jnp.float32)
                  + b2_ref[...])


def _final(u, e, x, w1e, b1, w2x, w2m, b2):
    n, d = x.shape
    e_total, de = e.shape
    k = e_total // n
    dout = w2x.shape[1]
    nb = 200
    body = functools.partial(_final_body, nb=nb, k=k, d=d)
    return pl.pallas_call(
        body,
        grid=(n // nb,),
        in_specs=[
            pl.BlockSpec((nb * k, d), lambda i: (i, 0)),
            pl.BlockSpec((nb * k, de), lambda i: (i, 0)),
            pl.BlockSpec((nb, d), lambda i: (i, 0)),
            pl.BlockSpec((de, d), lambda i: (0, 0)),
            pl.BlockSpec((1, d), lambda i: (0, 0)),
            pl.BlockSpec((d, dout), lambda i: (0, 0)),
            pl.BlockSpec((d, dout), lambda i: (0, 0)),
            pl.BlockSpec((1, dout), lambda i: (0, 0)),
        ],
        out_specs=pl.BlockSpec((nb, dout), lambda i: (i, 0)),
        out_shape=jax.ShapeDtypeStruct((n, dout), jnp.float32),
    )(u, e, x, w1e, b1, w2x, w2m, b2)


def kernel(x, edge_index, e, W1, b1, W2, b2):
    n, d = x.shape
    e_total = edge_index.shape[1]
    w1s = W1[:d]
    w1d = W1[d:2 * d]
    w1e = W1[2 * d:]
    w2x = W2[:d]
    w2m = W2[d:]
    src = edge_index[0].astype(jnp.int32)
    dst = edge_index[1].astype(jnp.int32)

    a, b = _proj(x, w1s, w1d)
    u = _make_sc_gather(e_total, n, d)(a, b, src, dst)
    return _final(u, e, x, w1e, b1.reshape(1, -1),
                  w2x, w2m, b2.reshape(1, -1))


# speedup vs baseline: 1.4868x; 1.4868x over previous
import functools

import jax
import jax.numpy as jnp
from jax import lax
from jax.experimental import pallas as pl
from jax.experimental.pallas import tpu as pltpu
from jax.experimental.pallas import tpu_sc as plsc

_NC = 2
_NS = 16
_NW = _NC * _NS
_CE = 80


def _proj_body(x_ref, ws_ref, wd_ref, a_ref, b_ref):
    x = x_ref[...]
    a_ref[...] = jnp.dot(x, ws_ref[...], preferred_element_type=jnp.float32)
    b_ref[...] = jnp.dot(x, wd_ref[...], preferred_element_type=jnp.float32)


def _proj(x, w1s, w1d):
    n, d = x.shape
    blk = 1000
    return pl.pallas_call(
        _proj_body,
        grid=(n // blk,),
        in_specs=[
            pl.BlockSpec((blk, d), lambda i: (i, 0)),
            pl.BlockSpec((d, d), lambda i: (0, 0)),
            pl.BlockSpec((d, d), lambda i: (0, 0)),
        ],
        out_specs=[
            pl.BlockSpec((blk, d), lambda i: (i, 0)),
            pl.BlockSpec((blk, d), lambda i: (i, 0)),
        ],
        out_shape=[jax.ShapeDtypeStruct((n, d), jnp.float32)] * 2,
    )(x, w1s, w1d)


@functools.lru_cache(maxsize=None)
def _make_sc_gather(e_total, n_nodes, d):
    epw = e_total // _NW
    nchunks = epw // _CE
    nvec = d // 16
    mesh = plsc.VectorSubcoreMesh(core_axis_name="c", subcore_axis_name="s",
                                  num_cores=_NC, num_subcores=_NS)

    @functools.partial(
        pl.kernel,
        out_type=jax.ShapeDtypeStruct((e_total, d), jnp.float32),
        mesh=mesh,
        scratch_types=[
            pltpu.VMEM((epw,), jnp.int32),
            pltpu.VMEM((epw,), jnp.int32),
            pltpu.VMEM((_CE, d), jnp.float32),
            pltpu.VMEM((_CE, d), jnp.float32),
            pltpu.VMEM((_CE, d), jnp.float32),
            pltpu.VMEM((_CE, d), jnp.float32),
            pltpu.SemaphoreType.DMA,
            pltpu.SemaphoreType.DMA,
        ],
    )
    def sc_fn(a_hbm, b_hbm, src_hbm, dst_hbm, u_hbm,
              src_v, dst_v, wa0, wb0, wa1, wb1, sem0, sem1):
        wid = lax.axis_index("s") * _NC + lax.axis_index("c")
        base = wid * epw

        pltpu.sync_copy(src_hbm.at[pl.ds(base, epw)], src_v)
        pltpu.sync_copy(dst_hbm.at[pl.ds(base, epw)], dst_v)

        def g_ops(c, wa, wb, sem):
            io = pl.multiple_of(c * _CE, 8)
            return (
                pltpu.make_async_copy(
                    a_hbm.at[src_v.at[pl.ds(io, _CE)]], wa, sem),
                pltpu.make_async_copy(
                    b_hbm.at[dst_v.at[pl.ds(io, _CE)]], wb, sem),
            )

        def g_start(c, wa, wb, sem):
            for cp in g_ops(c, wa, wb, sem):
                cp.start()

        def g_wait(c, wa, wb, sem):
            for cp in g_ops(c, wa, wb, sem):
                cp.wait()

        def compute(wa, wb):
            def row_body(r, rc):
                for v in range(nvec):
                    sl = pl.ds(v * 16, 16)
                    wa[r, sl] = wa[r, sl] + wb[r, sl]
                return rc

            lax.fori_loop(0, _CE, row_body, 0)

        def writeback(c, wa):
            off = pl.multiple_of(base + c * _CE, 8)
            pltpu.sync_copy(wa, u_hbm.at[pl.ds(off, _CE)])

        g_start(0, wa0, wb0, sem0)

        def pair_body(t, carry):
            c0 = t * 2
            g_wait(c0, wa0, wb0, sem0)
            g_start(c0 + 1, wa1, wb1, sem1)
            compute(wa0, wb0)
            writeback(c0, wa0)
            g_wait(c0 + 1, wa1, wb1, sem1)

            @pl.when(c0 + 2 < nchunks)
            def _():
                g_start(c0 + 2, wa0, wb0, sem0)

            compute(wa1, wb1)
            writeback(c0 + 1, wa1)
            return carry

        lax.fori_loop(0, nchunks // 2, pair_body, 0)

        if nchunks % 2 == 1:
            c = nchunks - 1
            g_wait(c, wa0, wb0, sem0)
            compute(wa0, wb0)
            writeback(c, wa0)

    return sc_fn


def _final_body(u_ref, e_ref, x_ref, w1e_ref, b1_ref,
                w2x_ref, w2m_ref, b2_ref, o_ref, *, nb, k, d):
    u = (u_ref[...]
         + jnp.dot(e_ref[...], w1e_ref[...], preferred_element_type=jnp.float32)
         + b1_ref[...])
    mess = jnp.where(u > 0, u, jnp.exp(jnp.minimum(u, 0.0)) - 1.0)
    am = jnp.mean(mess.reshape(nb, k, d), axis=1)
    o_ref[...] = (jnp.dot(x_ref[...], w2x_ref[...],
                          preferred_element_type=jnp.float32)
                  + jnp.dot(am, w2m_ref[...],
                            preferred_element_type=jnp.float32)
                  + b2_ref[...])


def _final(u, e, x, w1e, b1, w2x, w2m, b2):
    n, d = x.shape
    e_total, de = e.shape
    k = e_total // n
    dout = w2x.shape[1]
    nb = 200
    body = functools.partial(_final_body, nb=nb, k=k, d=d)
    return pl.pallas_call(
        body,
        grid=(n // nb,),
        in_specs=[
            pl.BlockSpec((nb * k, d), lambda i: (i, 0)),
            pl.BlockSpec((nb * k, de), lambda i: (i, 0)),
            pl.BlockSpec((nb, d), lambda i: (i, 0)),
            pl.BlockSpec((de, d), lambda i: (0, 0)),
            pl.BlockSpec((1, d), lambda i: (0, 0)),
            pl.BlockSpec((d, dout), lambda i: (0, 0)),
            pl.BlockSpec((d, dout), lambda i: (0, 0)),
            pl.BlockSpec((1, dout), lambda i: (0, 0)),
        ],
        out_specs=pl.BlockSpec((nb, dout), lambda i: (i, 0)),
        out_shape=jax.ShapeDtypeStruct((n, dout), jnp.float32),
    )(u, e, x, w1e, b1, w2x, w2m, b2)


def kernel(x, edge_index, e, W1, b1, W2, b2):
    n, d = x.shape
    e_total = edge_index.shape[1]
    w1s = W1[:d]
    w1d = W1[d:2 * d]
    w1e = W1[2 * d:]
    w2x = W2[:d]
    w2m = W2[d:]
    src = edge_index[0].astype(jnp.int32)
    dst = edge_index[1].astype(jnp.int32)

    a, b = _proj(x, w1s, w1d)
    u = _make_sc_gather(e_total, n, d)(a, b, src, dst)
    return _final(u, e, x, w1e, b1.reshape(1, -1),
                  w2x, w2m, b2.reshape(1, -1))


# speedup vs baseline: 1.4881x; 1.0009x over previous
import functools

import jax
import jax.numpy as jnp
from jax import lax
from jax.experimental import pallas as pl
from jax.experimental.pallas import tpu as pltpu
from jax.experimental.pallas import tpu_sc as plsc

_NC = 2
_NS = 16
_NW = _NC * _NS
_CE = 80


def _proj_body(x_ref, ws_ref, wd_ref, a_ref, b_ref):
    x = x_ref[...]
    a_ref[...] = jnp.dot(x, ws_ref[...], preferred_element_type=jnp.float32)
    b_ref[...] = jnp.dot(x, wd_ref[...], preferred_element_type=jnp.float32)


def _proj(x, w1s, w1d):
    n, d = x.shape
    blk = 1000
    return pl.pallas_call(
        _proj_body,
        grid=(n // blk,),
        in_specs=[
            pl.BlockSpec((blk, d), lambda i: (i, 0)),
            pl.BlockSpec((d, d), lambda i: (0, 0)),
            pl.BlockSpec((d, d), lambda i: (0, 0)),
        ],
        out_specs=[
            pl.BlockSpec((blk, d), lambda i: (i, 0)),
            pl.BlockSpec((blk, d), lambda i: (i, 0)),
        ],
        out_shape=[jax.ShapeDtypeStruct((n, d), jnp.float32)] * 2,
    )(x, w1s, w1d)


@functools.lru_cache(maxsize=None)
def _make_sc_gather(e_total, n_nodes, d):
    epw = e_total // _NW
    nchunks = epw // _CE
    nvec = d // 16
    mesh = plsc.VectorSubcoreMesh(core_axis_name="c", subcore_axis_name="s",
                                  num_cores=_NC, num_subcores=_NS)

    @functools.partial(
        pl.kernel,
        out_type=jax.ShapeDtypeStruct((e_total, d), jnp.float32),
        mesh=mesh,
        scratch_types=[
            pltpu.VMEM((epw,), jnp.int32),
            pltpu.VMEM((epw,), jnp.int32),
            pltpu.VMEM((_CE, d), jnp.float32),
            pltpu.VMEM((_CE, d), jnp.float32),
            pltpu.VMEM((_CE, d), jnp.float32),
            pltpu.VMEM((_CE, d), jnp.float32),
            pltpu.SemaphoreType.DMA,
            pltpu.SemaphoreType.DMA,
            pltpu.SemaphoreType.DMA,
            pltpu.SemaphoreType.DMA,
        ],
    )
    def sc_fn(a_hbm, b_hbm, src_hbm, dst_hbm, u_hbm,
              src_v, dst_v, wa0, wb0, wa1, wb1, sem0, sem1, wsem0, wsem1):
        wid = lax.axis_index("s") * _NC + lax.axis_index("c")
        base = wid * epw

        pltpu.sync_copy(src_hbm.at[pl.ds(base, epw)], src_v)
        pltpu.sync_copy(dst_hbm.at[pl.ds(base, epw)], dst_v)

        def g_ops(c, wa, wb, sem):
            io = pl.multiple_of(c * _CE, 8)
            return (
                pltpu.make_async_copy(
                    a_hbm.at[src_v.at[pl.ds(io, _CE)]], wa, sem),
                pltpu.make_async_copy(
                    b_hbm.at[dst_v.at[pl.ds(io, _CE)]], wb, sem),
            )

        def g_start(c, wa, wb, sem):
            for cp in g_ops(c, wa, wb, sem):
                cp.start()

        def g_wait(c, wa, wb, sem):
            for cp in g_ops(c, wa, wb, sem):
                cp.wait()

        def compute(wa, wb):
            def row_body(r, rc):
                for v in range(nvec):
                    sl = pl.ds(v * 16, 16)
                    wa[r, sl] = wa[r, sl] + wb[r, sl]
                return rc

            lax.fori_loop(0, _CE, row_body, 0)

        def wb_op(c, wa, wsem):
            off = pl.multiple_of(base + c * _CE, 8)
            return pltpu.make_async_copy(wa, u_hbm.at[pl.ds(off, _CE)], wsem)

        g_start(0, wa0, wb0, sem0)

        def pair_body(t, carry):
            c0 = t * 2
            g_wait(c0, wa0, wb0, sem0)

            @pl.when(c0 >= 1)
            def _():
                wb_op(c0 - 1, wa1, wsem1).wait()

            g_start(c0 + 1, wa1, wb1, sem1)
            compute(wa0, wb0)
            wb_op(c0, wa0, wsem0).start()
            g_wait(c0 + 1, wa1, wb1, sem1)
            wb_op(c0, wa0, wsem0).wait()

            @pl.when(c0 + 2 < nchunks)
            def _():
                g_start(c0 + 2, wa0, wb0, sem0)

            compute(wa1, wb1)
            wb_op(c0 + 1, wa1, wsem1).start()
            return carry

        lax.fori_loop(0, nchunks // 2, pair_body, 0)

        if nchunks % 2 == 1:
            c = nchunks - 1
            g_wait(c, wa0, wb0, sem0)
            wb_op(c - 1, wa1, wsem1).wait()
            compute(wa0, wb0)
            wb_op(c, wa0, wsem0).start()
            wb_op(c, wa0, wsem0).wait()
        else:
            wb_op(nchunks - 1, wa1, wsem1).wait()

    return sc_fn


def _final_body(u_ref, e_ref, x_ref, w1e_ref, b1_ref,
                w2x_ref, w2m_ref, b2_ref, o_ref, *, nb, k, d):
    u = (u_ref[...]
         + jnp.dot(e_ref[...], w1e_ref[...], preferred_element_type=jnp.float32)
         + b1_ref[...])
    mess = jnp.where(u > 0, u, jnp.exp(jnp.minimum(u, 0.0)) - 1.0)
    am = jnp.mean(mess.reshape(nb, k, d), axis=1)
    o_ref[...] = (jnp.dot(x_ref[...], w2x_ref[...],
                          preferred_element_type=jnp.float32)
                  + jnp.dot(am, w2m_ref[...],
                            preferred_element_type=jnp.float32)
                  + b2_ref[...])


def _final(u, e, x, w1e, b1, w2x, w2m, b2):
    n, d = x.shape
    e_total, de = e.shape
    k = e_total // n
    dout = w2x.shape[1]
    nb = 200
    body = functools.partial(_final_body, nb=nb, k=k, d=d)
    return pl.pallas_call(
        body,
        grid=(n // nb,),
        in_specs=[
            pl.BlockSpec((nb * k, d), lambda i: (i, 0)),
            pl.BlockSpec((nb * k, de), lambda i: (i, 0)),
            pl.BlockSpec((nb, d), lambda i: (i, 0)),
            pl.BlockSpec((de, d), lambda i: (0, 0)),
            pl.BlockSpec((1, d), lambda i: (0, 0)),
            pl.BlockSpec((d, dout), lambda i: (0, 0)),
            pl.BlockSpec((d, dout), lambda i: (0, 0)),
            pl.BlockSpec((1, dout), lambda i: (0, 0)),
        ],
        out_specs=pl.BlockSpec((nb, dout), lambda i: (i, 0)),
        out_shape=jax.ShapeDtypeStruct((n, dout), jnp.float32),
    )(u, e, x, w1e, b1, w2x, w2m, b2)


def kernel(x, edge_index, e, W1, b1, W2, b2):
    n, d = x.shape
    e_total = edge_index.shape[1]
    w1s = W1[:d]
    w1d = W1[d:2 * d]
    w1e = W1[2 * d:]
    w2x = W2[:d]
    w2m = W2[d:]
    src = edge_index[0].astype(jnp.int32)
    dst = edge_index[1].astype(jnp.int32)

    a, b = _proj(x, w1s, w1d)
    u = _make_sc_gather(e_total, n, d)(a, b, src, dst)
    return _final(u, e, x, w1e, b1.reshape(1, -1),
                  w2x, w2m, b2.reshape(1, -1))
